# baseline (speedup 1.0000x reference)
import functools

import jax
import jax.numpy as jnp
from jax import lax
from jax.experimental import pallas as pl
from jax.experimental.pallas import tpu as pltpu
from jax.experimental.pallas import tpu_sc as plsc

NC = 2
NS = 16
C = 128
K = 4

F32 = jnp.float32


def _ceil_to(x, m):
    return ((x + m - 1) // m) * m


@functools.lru_cache(maxsize=None)
def _sc_segsum(nblk, acc_n, e_pad, with_count, cb, depth):
    nblk_tot = nblk + (1 if with_count else 0)
    npass = -(-nblk_tot // NC)
    e_tile = e_pad // NS
    nch = e_tile // cb
    rows_t = acc_n // NS

    mesh = plsc.VectorSubcoreMesh(
        core_axis_name="c", subcore_axis_name="s",
        num_cores=NC, num_subcores=NS)

    scratch = [
        [pltpu.VMEM((cb,), jnp.int32) for _ in range(depth)],
        [pltpu.VMEM((cb,), jnp.int32) for _ in range(depth)],
        [pltpu.VMEM((cb, 128), F32) for _ in range(depth)],
        pltpu.VMEM_SHARED((acc_n, 128), F32),
        [pltpu.SemaphoreType.DMA for _ in range(depth)],
        pltpu.SemaphoreType.DMA,
    ]

    def body(table, srcb, dstp, zacc, ones_h, acc_out,
             sidx, didx, msgs, acc_sh, sems, semi):
        c = lax.axis_index("c")
        s = lax.axis_index("s")
        r0 = s * rows_t

        for p in range(npass):
            blk = p * NC + c
            ok = blk < nblk_tot

            @pl.when(ok)
            def _zero():
                pltpu.sync_copy(zacc.at[pl.ds(r0, rows_t)],
                                acc_sh.at[pl.ds(r0, rows_t)])

            plsc.subcore_barrier()

            @pl.when(blk < nblk)
            def _feature_panel():
                db = s * e_tile
                sb = blk * e_pad + db

                def idx_fetch(b, off):
                    pltpu.async_copy(srcb.at[pl.ds(sb + off, cb)],
                                     sidx[b], semi)
                    pltpu.async_copy(dstp.at[pl.ds(db + off, cb)],
                                     didx[b], semi)

                def idx_drain(b):
                    pltpu.make_async_copy(srcb.at[pl.ds(sb, cb)],
                                          sidx[b], semi).wait()
                    pltpu.make_async_copy(dstp.at[pl.ds(db, cb)],
                                          didx[b], semi).wait()

                def msg_drain(b):
                    pltpu.make_async_copy(table.at[pl.ds(0, cb)],
                                          msgs[b], sems[b]).wait()

                for b in range(depth - 1):
                    pltpu.sync_copy(srcb.at[pl.ds(sb + b * cb, cb)], sidx[b])
                    pltpu.sync_copy(dstp.at[pl.ds(db + b * cb, cb)], didx[b])
                    pltpu.async_copy(table.at[sidx[b]], msgs[b], sems[b])
                idx_fetch(depth - 1, (depth - 1) * cb)

                @pl.loop(0, nch // depth)
                def _ring(i):
                    for u in range(depth):
                        bg = (u + depth - 1) % depth
                        idx_drain(bg)
                        pltpu.async_copy(table.at[sidx[bg]], msgs[bg],
                                         sems[bg])
                        msg_drain(u)
                        pltpu.sync_copy(msgs[u], acc_sh.at[didx[u]],
                                        add=True)
                        idx_fetch(u, (i * depth + u + depth) * cb)

                idx_drain(depth - 1)
                for b in range(depth - 1):
                    msg_drain(b)

            if with_count:
                @pl.when(blk == nblk)
                def _count_panel():
                    db = s * e_tile
                    pltpu.sync_copy(ones_h, msgs[0])
                    pltpu.sync_copy(dstp.at[pl.ds(db, cb)], didx[0])

                    @pl.loop(0, nch // 2)
                    def _cpair(i):
                        pltpu.async_copy(
                            dstp.at[pl.ds(db + (2 * i + 1) * cb, cb)],
                            didx[1], semi)
                        pltpu.sync_copy(msgs[0], acc_sh.at[didx[0]], add=True)
                        pltpu.make_async_copy(dstp.at[pl.ds(db, cb)],
                                              didx[1], semi).wait()
                        pltpu.async_copy(
                            dstp.at[pl.ds(db + (2 * i + 2) * cb, cb)],
                            didx[0], semi)
                        pltpu.sync_copy(msgs[0], acc_sh.at[didx[1]], add=True)
                        pltpu.make_async_copy(dstp.at[pl.ds(db, cb)],
                                              didx[0], semi).wait()

            plsc.subcore_barrier()

            @pl.when(ok)
            def _writeback():
                pltpu.sync_copy(acc_sh.at[pl.ds(r0, rows_t)],
                                acc_out.at[blk, pl.ds(r0, rows_t)])

    return pl.kernel(
        body,
        out_type=jax.ShapeDtypeStruct((nblk_tot, acc_n, 128), F32),
        mesh=mesh, scratch_types=scratch)


def _block_table(x, nblk):
    n = x.shape[0]
    return x.reshape(n, nblk, 128).transpose(1, 0, 2).reshape(nblk * n, 128)


def _segmean_parts(x_src, ei, n_dst, with_count, cnt=None):
    n_src, d = x_src.shape
    nblk = d // 128
    e = ei.shape[1]
    acc_n = _ceil_to(n_dst + 1, NS * 8)

    cb = 128
    per_tile = (2_097_151 - acc_n * 128) // NS
    depth = 2
    for cand in (4, 3):
        if cand * (cb * 130) <= per_tile:
            depth = cand
            break
    e_pad = _ceil_to(e, NS * cb * depth)

    src = ei[0].astype(jnp.int32)
    dst = ei[1].astype(jnp.int32)
    pad = e_pad - e
    src_p = jnp.concatenate([src, jnp.zeros((pad,), jnp.int32)])
    dst_p = jnp.concatenate([dst, jnp.full((pad + depth * cb,), n_dst,
                                           jnp.int32)])
    offs = (jnp.arange(nblk, dtype=jnp.int32) * n_src)[:, None]
    srcb = jnp.concatenate([(src_p[None, :] + offs).reshape(-1),
                            jnp.zeros((depth * cb,), jnp.int32)])

    tbl = _block_table(x_src, nblk)
    zacc = jnp.zeros((acc_n, 128), F32)
    ones = jnp.ones((cb, 128), F32)

    k = _sc_segsum(nblk, acc_n, e_pad, with_count, cb, depth)
    acc = k(tbl, srcb, dst_p, zacc, ones)
    if with_count:
        return acc[:nblk, :n_dst, :], acc[nblk, :n_dst, :]
    return acc[:, :n_dst, :], cnt


def _sage_block_kernel(acc_ref, cnt_ref, x_ref, wl_ref, wr_ref, bl_ref, o_ref,
                       *, nblk):
    rcp = 1.0 / jnp.maximum(cnt_ref[...][:, 0:1], 1.0)
    y = jnp.dot(x_ref[...], wr_ref[...], preferred_element_type=F32)
    for b in range(nblk):
        y += jnp.dot(acc_ref[b] * rcp, wl_ref[b], preferred_element_type=F32)
    y += bl_ref[...]
    o_ref[...] = jnp.maximum(y, 0.0)


def _sage_relu(acc, cnt, x_dst, wlT, wrT, bl, rows_blk):
    nblk = acc.shape[0]
    n, d_dst = x_dst.shape
    h = wrT.shape[1]
    return pl.pallas_call(
        functools.partial(_sage_block_kernel, nblk=nblk),
        grid=(n // rows_blk,),
        in_specs=[
            pl.BlockSpec((nblk, rows_blk, 128), lambda i: (0, i, 0)),
            pl.BlockSpec((rows_blk, 128), lambda i: (i, 0)),
            pl.BlockSpec((rows_blk, d_dst), lambda i: (i, 0)),
            pl.BlockSpec((nblk, 128, h), lambda i: (0, 0, 0)),
            pl.BlockSpec((d_dst, h), lambda i: (0, 0)),
            pl.BlockSpec((1, h), lambda i: (0, 0)),
        ],
        out_specs=pl.BlockSpec((rows_blk, h), lambda i: (i, 0)),
        out_shape=jax.ShapeDtypeStruct((n, h), F32),
    )(acc, cnt, x_dst, wlT, wrT, bl)


def _dom_kernel(acc_ud_ref, cnt_ud_ref, acc_td_ref, cnt_td_ref, x_ref,
                wl_ud_ref, wl_td_ref, wr_ref, b_ref, o_ref):
    rcp_ud = 1.0 / jnp.maximum(cnt_ud_ref[...][:, 0:1], 1.0)
    rcp_td = 1.0 / jnp.maximum(cnt_td_ref[...][:, 0:1], 1.0)
    y = jnp.dot(x_ref[...], wr_ref[...], preferred_element_type=F32)
    for b in range(4):
        y += jnp.dot(acc_ud_ref[b] * rcp_ud, wl_ud_ref[b],
                     preferred_element_type=F32)
    y += jnp.dot(acc_td_ref[0] * rcp_td, wl_td_ref[0],
                 preferred_element_type=F32)
    y += b_ref[...]
    o_ref[...] = jnp.maximum(y * 0.5, 0.0)


def _head_kernel(acc_ref, cnt_ref, hu_ref, wl_ref, wr_ref, bl_ref,
                 l1_ref, b1_ref, l2_ref, b2_ref, o_ref):
    rcp = 1.0 / jnp.maximum(cnt_ref[...][:, 0:1], 1.0)
    z = jnp.dot(hu_ref[...], wr_ref[...], preferred_element_type=F32)
    for b in range(4):
        z += jnp.dot(acc_ref[b] * rcp, wl_ref[b], preferred_element_type=F32)
    z = jnp.maximum(z + bl_ref[...], 0.0)
    x = jnp.maximum(jnp.dot(z, l1_ref[...], preferred_element_type=F32)
                    + b1_ref[...], 0.0)
    logits = jnp.dot(x, l2_ref[...], preferred_element_type=F32) + b2_ref[...]
    m = jnp.max(logits, axis=1, keepdims=True)
    e = jnp.exp(logits - m)
    o_ref[...] = e / jnp.sum(e, axis=1, keepdims=True)


def kernel(x_url, x_domain, x_tld, ei_ud, ei_du, ei_dt, ei_td, params):
    p = params
    n_url, d_url = x_url.shape
    n_dom, d_dom = x_domain.shape
    h = p["lin1_W"].shape[1]

    acc_du, cnt_du = _segmean_parts(x_domain, ei_du, n_url, True)
    acc_ud, cnt_ud = _segmean_parts(x_url, ei_ud, n_dom, True)
    acc_td, cnt_td = _segmean_parts(x_tld, ei_td, n_dom, True)

    h_url = _sage_relu(
        acc_du, cnt_du, x_url,
        p["c1_du_Wl"].T.reshape(d_dom // 128, 128, h),
        p["c1_du_Wr"].T, p["c1_du_bl"].reshape(1, h), 1000)

    wr_sum = p["c1_ud_Wr"].T + p["c1_td_Wr"].T
    b_sum = (p["c1_ud_bl"] + p["c1_td_bl"]).reshape(1, h)
    h_dom = pl.pallas_call(
        _dom_kernel,
        out_shape=jax.ShapeDtypeStruct((n_dom, h), F32),
    )(acc_ud, cnt_ud, acc_td, cnt_td, x_domain,
      p["c1_ud_Wl"].T.reshape(4, 128, h),
      p["c1_td_Wl"].T.reshape(1, 128, h),
      wr_sum, b_sum)

    acc2, _ = _segmean_parts(h_dom, ei_du, n_url, False, cnt_du)

    out = pl.pallas_call(
        _head_kernel,
        grid=(n_url // 1000,),
        in_specs=[
            pl.BlockSpec((4, 1000, 128), lambda i: (0, i, 0)),
            pl.BlockSpec((1000, 128), lambda i: (i, 0)),
            pl.BlockSpec((1000, h), lambda i: (i, 0)),
            pl.BlockSpec((4, 128, h), lambda i: (0, 0, 0)),
            pl.BlockSpec((h, h), lambda i: (0, 0)),
            pl.BlockSpec((1, h), lambda i: (0, 0)),
            pl.BlockSpec((h, h), lambda i: (0, 0)),
            pl.BlockSpec((1, h), lambda i: (0, 0)),
            pl.BlockSpec((h, 16), lambda i: (0, 0)),
            pl.BlockSpec((1, 16), lambda i: (0, 0)),
        ],
        out_specs=pl.BlockSpec((1000, 16), lambda i: (i, 0)),
        out_shape=jax.ShapeDtypeStruct((n_url, 16), F32),
    )(acc2, cnt_du, h_url,
      p["c2_du_Wl"].T.reshape(4, 128, h), p["c2_du_Wr"].T,
      p["c2_du_bl"].reshape(1, h),
      p["lin1_W"].T, p["lin1_b"].reshape(1, h),
      p["lin2_W"].T, p["lin2_b"].reshape(1, 16))
    return out


# speedup vs baseline: 1.5301x; 1.5301x over previous
import functools

import jax
import jax.numpy as jnp
from jax import lax
from jax.experimental import pallas as pl
from jax.experimental.pallas import tpu as pltpu
from jax.experimental.pallas import tpu_sc as plsc

NC = 2
NS = 16
C = 128
K = 4

F32 = jnp.float32


def _ceil_to(x, m):
    return ((x + m - 1) // m) * m


@functools.lru_cache(maxsize=None)
def _sc_segsum(nblk, acc_n, e_pad, with_count, cb, depth):
    nblk_tot = nblk + (1 if with_count else 0)
    npass = -(-nblk_tot // NC)
    e_tile = e_pad // NS
    nch = e_tile // cb
    rows_t = acc_n // NS

    mesh = plsc.VectorSubcoreMesh(
        core_axis_name="c", subcore_axis_name="s",
        num_cores=NC, num_subcores=NS)

    scratch = [
        [pltpu.VMEM((cb,), jnp.int32) for _ in range(depth)],
        [pltpu.VMEM((cb,), jnp.int32) for _ in range(depth)],
        [pltpu.VMEM((cb, 128), F32) for _ in range(depth)],
        pltpu.VMEM_SHARED((acc_n, 128), F32),
        [pltpu.SemaphoreType.DMA for _ in range(depth)],
        pltpu.SemaphoreType.DMA,
    ]

    def body(table, srcb, dstp, zacc, ones_h, acc_out,
             sidx, didx, msgs, acc_sh, sems, semi):
        c = lax.axis_index("c")
        s = lax.axis_index("s")
        r0 = s * rows_t

        for p in range(npass):
            blk = p * NC + c
            ok = blk < nblk_tot

            @pl.when(ok)
            def _zero():
                pltpu.sync_copy(zacc.at[pl.ds(r0, rows_t)],
                                acc_sh.at[pl.ds(r0, rows_t)])

            plsc.subcore_barrier()

            @pl.when(blk < nblk)
            def _feature_panel():
                db = s * e_tile
                sb = blk * e_pad + db

                def idx_fetch(b, off):
                    pltpu.async_copy(srcb.at[pl.ds(sb + off, cb)],
                                     sidx[b], semi)
                    pltpu.async_copy(dstp.at[pl.ds(db + off, cb)],
                                     didx[b], semi)

                def idx_drain(b):
                    pltpu.make_async_copy(srcb.at[pl.ds(sb, cb)],
                                          sidx[b], semi).wait()
                    pltpu.make_async_copy(dstp.at[pl.ds(db, cb)],
                                          didx[b], semi).wait()

                def msg_drain(b):
                    pltpu.make_async_copy(table.at[pl.ds(0, cb)],
                                          msgs[b], sems[b]).wait()

                for b in range(depth - 1):
                    pltpu.sync_copy(srcb.at[pl.ds(sb + b * cb, cb)], sidx[b])
                    pltpu.sync_copy(dstp.at[pl.ds(db + b * cb, cb)], didx[b])
                    pltpu.async_copy(table.at[sidx[b]], msgs[b], sems[b])
                idx_fetch(depth - 1, (depth - 1) * cb)

                @pl.loop(0, nch // depth)
                def _ring(i):
                    for u in range(depth):
                        bg = (u + depth - 1) % depth
                        idx_drain(bg)
                        pltpu.async_copy(table.at[sidx[bg]], msgs[bg],
                                         sems[bg])
                        msg_drain(u)
                        pltpu.sync_copy(msgs[u], acc_sh.at[didx[u]],
                                        add=True)
                        idx_fetch(u, (i * depth + u + depth) * cb)

                idx_drain(depth - 1)
                for b in range(depth - 1):
                    msg_drain(b)

            if with_count:
                @pl.when(blk == nblk)
                def _count_panel():
                    db = s * e_tile
                    pltpu.sync_copy(ones_h, msgs[0])
                    pltpu.sync_copy(dstp.at[pl.ds(db, cb)], didx[0])

                    @pl.loop(0, nch // 2)
                    def _cpair(i):
                        pltpu.async_copy(
                            dstp.at[pl.ds(db + (2 * i + 1) * cb, cb)],
                            didx[1], semi)
                        pltpu.sync_copy(msgs[0], acc_sh.at[didx[0]], add=True)
                        pltpu.make_async_copy(dstp.at[pl.ds(db, cb)],
                                              didx[1], semi).wait()
                        pltpu.async_copy(
                            dstp.at[pl.ds(db + (2 * i + 2) * cb, cb)],
                            didx[0], semi)
                        pltpu.sync_copy(msgs[0], acc_sh.at[didx[1]], add=True)
                        pltpu.make_async_copy(dstp.at[pl.ds(db, cb)],
                                              didx[0], semi).wait()

            plsc.subcore_barrier()

            @pl.when(ok)
            def _writeback():
                pltpu.sync_copy(acc_sh.at[pl.ds(r0, rows_t)],
                                acc_out.at[blk, pl.ds(r0, rows_t)])

    return pl.kernel(
        body,
        out_type=jax.ShapeDtypeStruct((nblk_tot, acc_n, 128), F32),
        mesh=mesh, scratch_types=scratch)


def _block_table(x, nblk):
    n = x.shape[0]
    return x.reshape(n, nblk, 128).transpose(1, 0, 2).reshape(nblk * n, 128)


def _segmean_parts(x_src, ei, n_dst, with_count, cnt=None):
    n_src, d = x_src.shape
    nblk = d // 128
    e = ei.shape[1]
    acc_n = _ceil_to(n_dst + 1, NS * 8)

    cb = 128
    depth = 2
    e_pad = _ceil_to(e, NS * cb * depth)

    src = ei[0].astype(jnp.int32)
    dst = ei[1].astype(jnp.int32)
    pad = e_pad - e
    src_p = jnp.concatenate([src, jnp.zeros((pad,), jnp.int32)])
    dst_p = jnp.concatenate([dst, jnp.full((pad + depth * cb,), n_dst,
                                           jnp.int32)])
    offs = (jnp.arange(nblk, dtype=jnp.int32) * n_src)[:, None]
    srcb = jnp.concatenate([(src_p[None, :] + offs).reshape(-1),
                            jnp.zeros((depth * cb,), jnp.int32)])

    tbl = _block_table(x_src, nblk)
    zacc = jnp.zeros((acc_n, 128), F32)
    ones = jnp.ones((cb, 128), F32)

    k = _sc_segsum(nblk, acc_n, e_pad, with_count, cb, depth)
    acc = k(tbl, srcb, dst_p, zacc, ones)
    if with_count:
        return acc[:nblk, :n_dst, :], acc[nblk, :n_dst, :]
    return acc[:, :n_dst, :], cnt


def _sage_block_kernel(acc_ref, cnt_ref, x_ref, wl_ref, wr_ref, bl_ref, o_ref,
                       *, nblk):
    rcp = 1.0 / jnp.maximum(cnt_ref[...][:, 0:1], 1.0)
    y = jnp.dot(x_ref[...], wr_ref[...], preferred_element_type=F32)
    for b in range(nblk):
        y += jnp.dot(acc_ref[b] * rcp, wl_ref[b], preferred_element_type=F32)
    y += bl_ref[...]
    o_ref[...] = jnp.maximum(y, 0.0)


def _sage_relu(acc, cnt, x_dst, wlT, wrT, bl, rows_blk):
    nblk = acc.shape[0]
    n, d_dst = x_dst.shape
    h = wrT.shape[1]
    return pl.pallas_call(
        functools.partial(_sage_block_kernel, nblk=nblk),
        grid=(n // rows_blk,),
        in_specs=[
            pl.BlockSpec((nblk, rows_blk, 128), lambda i: (0, i, 0)),
            pl.BlockSpec((rows_blk, 128), lambda i: (i, 0)),
            pl.BlockSpec((rows_blk, d_dst), lambda i: (i, 0)),
            pl.BlockSpec((nblk, 128, h), lambda i: (0, 0, 0)),
            pl.BlockSpec((d_dst, h), lambda i: (0, 0)),
            pl.BlockSpec((1, h), lambda i: (0, 0)),
        ],
        out_specs=pl.BlockSpec((rows_blk, h), lambda i: (i, 0)),
        out_shape=jax.ShapeDtypeStruct((n, h), F32),
    )(acc, cnt, x_dst, wlT, wrT, bl)


def _dom_kernel(acc_ud_ref, cnt_ud_ref, acc_td_ref, cnt_td_ref, x_ref,
                wl_ud_ref, wl_td_ref, wr_ref, b_ref, o_ref):
    rcp_ud = 1.0 / jnp.maximum(cnt_ud_ref[...][:, 0:1], 1.0)
    rcp_td = 1.0 / jnp.maximum(cnt_td_ref[...][:, 0:1], 1.0)
    y = jnp.dot(x_ref[...], wr_ref[...], preferred_element_type=F32)
    for b in range(4):
        y += jnp.dot(acc_ud_ref[b] * rcp_ud, wl_ud_ref[b],
                     preferred_element_type=F32)
    y += jnp.dot(acc_td_ref[0] * rcp_td, wl_td_ref[0],
                 preferred_element_type=F32)
    y += b_ref[...]
    o_ref[...] = jnp.maximum(y * 0.5, 0.0)


def _head_kernel(acc_ref, cnt_ref, hu_ref, wl_ref, wr_ref, bl_ref,
                 l1_ref, b1_ref, l2_ref, b2_ref, o_ref):
    rcp = 1.0 / jnp.maximum(cnt_ref[...][:, 0:1], 1.0)
    z = jnp.dot(hu_ref[...], wr_ref[...], preferred_element_type=F32)
    for b in range(4):
        z += jnp.dot(acc_ref[b] * rcp, wl_ref[b], preferred_element_type=F32)
    z = jnp.maximum(z + bl_ref[...], 0.0)
    x = jnp.maximum(jnp.dot(z, l1_ref[...], preferred_element_type=F32)
                    + b1_ref[...], 0.0)
    logits = jnp.dot(x, l2_ref[...], preferred_element_type=F32) + b2_ref[...]
    m = jnp.max(logits, axis=1, keepdims=True)
    e = jnp.exp(logits - m)
    o_ref[...] = e / jnp.sum(e, axis=1, keepdims=True)


def kernel(x_url, x_domain, x_tld, ei_ud, ei_du, ei_dt, ei_td, params):
    p = params
    n_url, d_url = x_url.shape
    n_dom, d_dom = x_domain.shape
    h = p["lin1_W"].shape[1]

    acc_du, cnt_du = _segmean_parts(x_domain, ei_du, n_url, True)
    acc_ud, cnt_ud = _segmean_parts(x_url, ei_ud, n_dom, True)
    acc_td, cnt_td = _segmean_parts(x_tld, ei_td, n_dom, True)

    h_url = _sage_relu(
        acc_du, cnt_du, x_url,
        p["c1_du_Wl"].T.reshape(d_dom // 128, 128, h),
        p["c1_du_Wr"].T, p["c1_du_bl"].reshape(1, h), 1000)

    wr_sum = p["c1_ud_Wr"].T + p["c1_td_Wr"].T
    b_sum = (p["c1_ud_bl"] + p["c1_td_bl"]).reshape(1, h)
    h_dom = pl.pallas_call(
        _dom_kernel,
        out_shape=jax.ShapeDtypeStruct((n_dom, h), F32),
    )(acc_ud, cnt_ud, acc_td, cnt_td, x_domain,
      p["c1_ud_Wl"].T.reshape(4, 128, h),
      p["c1_td_Wl"].T.reshape(1, 128, h),
      wr_sum, b_sum)

    acc2, _ = _segmean_parts(h_dom, ei_du, n_url, False, cnt_du)

    out = pl.pallas_call(
        _head_kernel,
        grid=(n_url // 1000,),
        in_specs=[
            pl.BlockSpec((4, 1000, 128), lambda i: (0, i, 0)),
            pl.BlockSpec((1000, 128), lambda i: (i, 0)),
            pl.BlockSpec((1000, h), lambda i: (i, 0)),
            pl.BlockSpec((4, 128, h), lambda i: (0, 0, 0)),
            pl.BlockSpec((h, h), lambda i: (0, 0)),
            pl.BlockSpec((1, h), lambda i: (0, 0)),
            pl.BlockSpec((h, h), lambda i: (0, 0)),
            pl.BlockSpec((1, h), lambda i: (0, 0)),
            pl.BlockSpec((h, 16), lambda i: (0, 0)),
            pl.BlockSpec((1, 16), lambda i: (0, 0)),
        ],
        out_specs=pl.BlockSpec((1000, 16), lambda i: (i, 0)),
        out_shape=jax.ShapeDtypeStruct((n_url, 16), F32),
    )(acc2, cnt_du, h_url,
      p["c2_du_Wl"].T.reshape(4, 128, h), p["c2_du_Wr"].T,
      p["c2_du_bl"].reshape(1, h),
      p["lin1_W"].T, p["lin1_b"].reshape(1, h),
      p["lin2_W"].T, p["lin2_b"].reshape(1, 16))
    return out
